# baseline (speedup 1.0000x reference)
import functools

import jax
import jax.numpy as jnp
from jax import lax
from jax.experimental import pallas as pl
from jax.experimental.pallas import tpu as pltpu, tpu_sc as plsc

N = 10000
E = 320000
D = 128
ALPHA = 0.2

NC = 2
NS = 16
NW = NC * NS
EW = E // NW
CHUNK = 80
NCHUNK_W = EW // CHUNK
SBLK = 5
SB = NCHUNK_W // SBLK
NP = 10240
RPT = NP // NS
LANES = 16


def _matmul_body(h_ref, w_ref, o_ref):
    o_ref[...] = lax.dot_general(
        h_ref[...], w_ref[...], (((1,), (1,)), ((), ())),
        preferred_element_type=jnp.float32)


def _matmul(h, W):
    BM = 2000
    return pl.pallas_call(
        _matmul_body,
        grid=(N // BM,),
        in_specs=[pl.BlockSpec((BM, D), lambda i: (i, 0)),
                  pl.BlockSpec((D, D), lambda i: (0, 0))],
        out_specs=pl.BlockSpec((BM, D), lambda i: (i, 0)),
        out_shape=jax.ShapeDtypeStruct((N, D), jnp.float32),
    )(h, W)


_mesh = plsc.VectorSubcoreMesh(core_axis_name="c", subcore_axis_name="s")


@functools.partial(
    pl.kernel,
    out_type=jax.ShapeDtypeStruct((NC, NP, D), jnp.float32),
    mesh=_mesh,
    scratch_types=[
        pltpu.VMEM((SB, CHUNK), jnp.int32),
        pltpu.VMEM((SB, CHUNK), jnp.int32),
        pltpu.VMEM((SB, CHUNK), jnp.float32),
        pltpu.VMEM((CHUNK, D), jnp.float32),
        pltpu.VMEM((CHUNK, D), jnp.float32),
        pltpu.VMEM_SHARED((NP, D), jnp.float32),
        pltpu.SemaphoreType.DMA,
        pltpu.SemaphoreType.DMA,
        pltpu.SemaphoreType.DMA,
        pltpu.SemaphoreType.DMA,
        pltpu.SemaphoreType.DMA,
        pltpu.SemaphoreType.DMA,
        pltpu.SemaphoreType.DMA,
        pltpu.SemaphoreType.DMA,
        pltpu.SemaphoreType.DMA,
        pltpu.SemaphoreType.DMA,
    ],
)
def _edge_kernel(hlin, row4, col4, w4, out, rowv, colv, wv,
                 gbufA, gbufB, acc,
                 gA0, gA1, gA2, gA3, gB0, gB1, gB2, gB3, semsA, semsB):
    c = lax.axis_index("c")
    s = lax.axis_index("s")
    wid = c * NS + s

    gA = (gA0, gA1, gA2, gA3)
    gB = (gB0, gB1, gB2, gB3)
    QS = ((0, 24), (24, 24), (48, 16), (64, 16))

    def drain(sem, buf):
        pltpu.make_async_copy(hlin.at[pl.ds(0, CHUNK)], buf, sem).wait()

    def drain_quarters(sems, buf):
        for q, (off, sz) in enumerate(QS):
            pltpu.make_async_copy(
                hlin.at[pl.ds(0, sz)], buf.at[pl.ds(off, sz)], sems[q]).wait()

    def gather4(k, buf, sems):
        for q, (off, sz) in enumerate(QS):
            pltpu.async_copy(
                hlin.at[colv.at[k, pl.ds(off, sz)]],
                buf.at[pl.ds(off, sz)], sems[q])

    zero = jnp.zeros((LANES,), jnp.float32)

    def zbody(r, _):
        for j in range(D // LANES):
            gbufA[r, pl.ds(j * LANES, LANES)] = zero
        return 0

    lax.fori_loop(0, CHUNK, zbody, 0)
    for i in range(RPT // CHUNK):
        pltpu.sync_copy(gbufA, acc.at[pl.ds(s * RPT + i * CHUNK, CHUNK)])
    plsc.subcore_barrier()

    def scale(buf, k):
        for g in range(CHUNK // LANES):
            wvec = wv[k, pl.ds(g * LANES, LANES)]
            for i in range(LANES):
                w = wvec[i]
                e = g * LANES + i
                for j in range(D // LANES):
                    buf[e, pl.ds(j * LANES, LANES)] = (
                        buf[e, pl.ds(j * LANES, LANES)] * w)

    def step(k, X, gX, sX, Y, gY, sY):
        @pl.when(k >= 1)
        def _():
            drain(sY, Y)

        @pl.when(k <= SB - 2)
        def _():
            gather4(k + 1, Y, gY)

        drain_quarters(gX, X)
        scale(X, k)
        pltpu.async_copy(X, acc.at[rowv.at[k]], sX, add=True)

    def sblock(b, _):
        pltpu.sync_copy(col4.at[wid, b], colv)
        gather4(0, gbufA, gA)
        pltpu.sync_copy(row4.at[wid, b], rowv)
        pltpu.sync_copy(w4.at[wid, b], wv)

        def chunk(k, _):
            @pl.when(k % 2 == 0)
            def _():
                step(k, gbufA, gA, semsA, gbufB, gB, semsB)

            @pl.when(k % 2 == 1)
            def _():
                step(k, gbufB, gB, semsB, gbufA, gA, semsA)

            return 0

        lax.fori_loop(0, SB, chunk, 0)
        drain(semsA, gbufA)
        return 0

    lax.fori_loop(0, SBLK, sblock, 0)

    plsc.subcore_barrier()
    pltpu.sync_copy(acc.at[pl.ds(s * RPT, RPT)], out.at[c, pl.ds(s * RPT, RPT)])


def _finish_body(p_ref, h0_ref, g_ref, b_ref, o_ref):
    a = p_ref[0] + p_ref[1]
    mean = jnp.mean(a, axis=-1, keepdims=True)
    xc = a - mean
    var = jnp.mean(xc * xc, axis=-1, keepdims=True)
    y = xc * lax.rsqrt(var + 1e-5) * g_ref[...] + b_ref[...]
    y = jnp.maximum(y, 0.0)
    o_ref[...] = (1.0 - ALPHA) * y + ALPHA * h0_ref[...]


def _finish(partials, h0, gamma, beta):
    BM = 5000
    return pl.pallas_call(
        _finish_body,
        grid=(N // BM,),
        in_specs=[pl.BlockSpec((NC, BM, D), lambda i: (0, i, 0)),
                  pl.BlockSpec((BM, D), lambda i: (i, 0)),
                  pl.BlockSpec((1, D), lambda i: (0, 0)),
                  pl.BlockSpec((1, D), lambda i: (0, 0))],
        out_specs=pl.BlockSpec((BM, D), lambda i: (i, 0)),
        out_shape=jax.ShapeDtypeStruct((N, D), jnp.float32),
    )(partials, h0, gamma, beta)


def kernel(h, h0, row, col, norm_weight, W, gamma, beta):
    row4 = row.astype(jnp.int32).reshape(NW, SBLK, SB, CHUNK)
    col4 = col.astype(jnp.int32).reshape(NW, SBLK, SB, CHUNK)
    w4 = norm_weight.reshape(NW, SBLK, SB, CHUNK)
    h_lin = _matmul(h, W)
    partials = _edge_kernel(h_lin, row4, col4, w4)
    return _finish(partials, h0, gamma.reshape(1, D), beta.reshape(1, D))


# speedup vs baseline: 1.0053x; 1.0053x over previous
import functools

import jax
import jax.numpy as jnp
from jax import lax
from jax.experimental import pallas as pl
from jax.experimental.pallas import tpu as pltpu, tpu_sc as plsc

N = 10000
E = 320000
D = 128
ALPHA = 0.2

NC = 2
NS = 16
NW = NC * NS
EW = E // NW
CHUNK = 80
NCHUNK_W = EW // CHUNK
SBLK = 5
SB = NCHUNK_W // SBLK
NP = 10240
RPT = NP // NS
LANES = 16


def _matmul_body(h_ref, w_ref, o_ref):
    o_ref[...] = lax.dot_general(
        h_ref[...], w_ref[...], (((1,), (1,)), ((), ())),
        preferred_element_type=jnp.float32)


def _matmul(h, W):
    BM = 2000
    return pl.pallas_call(
        _matmul_body,
        grid=(N // BM,),
        in_specs=[pl.BlockSpec((BM, D), lambda i: (i, 0)),
                  pl.BlockSpec((D, D), lambda i: (0, 0))],
        out_specs=pl.BlockSpec((BM, D), lambda i: (i, 0)),
        out_shape=jax.ShapeDtypeStruct((N, D), jnp.float32),
    )(h, W)


_mesh = plsc.VectorSubcoreMesh(core_axis_name="c", subcore_axis_name="s")


@functools.partial(
    pl.kernel,
    out_type=jax.ShapeDtypeStruct((NC, NP, D), jnp.float32),
    mesh=_mesh,
    scratch_types=[
        pltpu.VMEM((SB, CHUNK), jnp.int32),
        pltpu.VMEM((SB, CHUNK), jnp.int32),
        pltpu.VMEM((SB, CHUNK), jnp.float32),
        pltpu.VMEM((CHUNK, D), jnp.float32),
        pltpu.VMEM((CHUNK, D), jnp.float32),
        pltpu.VMEM_SHARED((NP, D), jnp.float32),
        pltpu.SemaphoreType.DMA,
        pltpu.SemaphoreType.DMA,
        pltpu.SemaphoreType.DMA,
        pltpu.SemaphoreType.DMA,
        pltpu.SemaphoreType.DMA,
        pltpu.SemaphoreType.DMA,
    ],
)
def _edge_kernel(hlin, row4, col4, w4, out, rowv, colv, wv,
                 gbufA, gbufB, acc, semgA, semgA2, semgB, semgB2, semsA, semsB):
    c = lax.axis_index("c")
    s = lax.axis_index("s")
    wid = c * NS + s

    H = CHUNK // 2

    def drain(sem, buf):
        pltpu.make_async_copy(hlin.at[pl.ds(0, CHUNK)], buf, sem).wait()

    def drain_half(sem, buf):
        pltpu.make_async_copy(
            hlin.at[pl.ds(0, H)], buf.at[pl.ds(0, H)], sem).wait()

    def gather2(k, buf, sem_lo, sem_hi):
        pltpu.async_copy(
            hlin.at[colv.at[k, pl.ds(0, H)]], buf.at[pl.ds(0, H)], sem_lo)
        pltpu.async_copy(
            hlin.at[colv.at[k, pl.ds(H, H)]], buf.at[pl.ds(H, H)], sem_hi)

    zero = jnp.zeros((LANES,), jnp.float32)

    def zbody(r, _):
        for j in range(D // LANES):
            gbufA[r, pl.ds(j * LANES, LANES)] = zero
        return 0

    lax.fori_loop(0, CHUNK, zbody, 0)
    for i in range(RPT // CHUNK):
        pltpu.sync_copy(gbufA, acc.at[pl.ds(s * RPT + i * CHUNK, CHUNK)])
    plsc.subcore_barrier()

    def scale(buf, k):
        for g in range(CHUNK // LANES):
            wvec = wv[k, pl.ds(g * LANES, LANES)]
            for i in range(LANES):
                w = wvec[i]
                e = g * LANES + i
                for j in range(D // LANES):
                    buf[e, pl.ds(j * LANES, LANES)] = (
                        buf[e, pl.ds(j * LANES, LANES)] * w)

    def step(k, X, gX, gX2, sX, Y, gY, gY2, sY):
        @pl.when(k >= 1)
        def _():
            drain(sY, Y)

        @pl.when(k <= SB - 2)
        def _():
            gather2(k + 1, Y, gY, gY2)

        drain_half(gX, X)
        drain_half(gX2, X)
        scale(X, k)
        pltpu.async_copy(X, acc.at[rowv.at[k]], sX, add=True)

    def sblock(b, _):
        pltpu.sync_copy(col4.at[wid, b], colv)
        gather2(0, gbufA, semgA, semgA2)
        pltpu.sync_copy(row4.at[wid, b], rowv)
        pltpu.sync_copy(w4.at[wid, b], wv)

        def chunk(k, _):
            @pl.when(k % 2 == 0)
            def _():
                step(k, gbufA, semgA, semgA2, semsA, gbufB, semgB, semgB2, semsB)

            @pl.when(k % 2 == 1)
            def _():
                step(k, gbufB, semgB, semgB2, semsB, gbufA, semgA, semgA2, semsA)

            return 0

        lax.fori_loop(0, SB, chunk, 0)
        drain(semsA, gbufA)
        return 0

    lax.fori_loop(0, SBLK, sblock, 0)

    plsc.subcore_barrier()
    pltpu.sync_copy(acc.at[pl.ds(s * RPT, RPT)], out.at[c, pl.ds(s * RPT, RPT)])


def _finish_body(p_ref, h0_ref, g_ref, b_ref, o_ref):
    a = p_ref[0] + p_ref[1]
    mean = jnp.mean(a, axis=-1, keepdims=True)
    xc = a - mean
    var = jnp.mean(xc * xc, axis=-1, keepdims=True)
    y = xc * lax.rsqrt(var + 1e-5) * g_ref[...] + b_ref[...]
    y = jnp.maximum(y, 0.0)
    o_ref[...] = (1.0 - ALPHA) * y + ALPHA * h0_ref[...]


def _finish(partials, h0, gamma, beta):
    BM = 10000
    return pl.pallas_call(
        _finish_body,
        grid=(N // BM,),
        in_specs=[pl.BlockSpec((NC, BM, D), lambda i: (0, i, 0)),
                  pl.BlockSpec((BM, D), lambda i: (i, 0)),
                  pl.BlockSpec((1, D), lambda i: (0, 0)),
                  pl.BlockSpec((1, D), lambda i: (0, 0))],
        out_specs=pl.BlockSpec((BM, D), lambda i: (i, 0)),
        out_shape=jax.ShapeDtypeStruct((N, D), jnp.float32),
    )(partials, h0, gamma, beta)


def kernel(h, h0, row, col, norm_weight, W, gamma, beta):
    row4 = row.astype(jnp.int32).reshape(NW, SBLK, SB, CHUNK)
    col4 = col.astype(jnp.int32).reshape(NW, SBLK, SB, CHUNK)
    w4 = norm_weight.reshape(NW, SBLK, SB, CHUNK)
    h_lin = _matmul(h, W)
    partials = _edge_kernel(h_lin, row4, col4, w4)
    return _finish(partials, h0, gamma.reshape(1, D), beta.reshape(1, D))


# speedup vs baseline: 1.0282x; 1.0227x over previous
import functools

import jax
import jax.numpy as jnp
from jax import lax
from jax.experimental import pallas as pl
from jax.experimental.pallas import tpu as pltpu, tpu_sc as plsc

N = 10000
E = 320000
D = 128
ALPHA = 0.2

NC = 2
NS = 16
NW = NC * NS
EW = E // NW
CHUNK = 80
NCHUNK_W = EW // CHUNK
SBLK = 5
SB = NCHUNK_W // SBLK
NP = 10240
RPT = NP // NS
LANES = 16


def _matmul_body(h_ref, w_ref, o_ref):
    o_ref[...] = lax.dot_general(
        h_ref[...], w_ref[...], (((1,), (1,)), ((), ())),
        preferred_element_type=jnp.float32)


def _matmul(h, W):
    BM = 5000
    return pl.pallas_call(
        _matmul_body,
        grid=(N // BM,),
        in_specs=[pl.BlockSpec((BM, D), lambda i: (i, 0)),
                  pl.BlockSpec((D, D), lambda i: (0, 0))],
        out_specs=pl.BlockSpec((BM, D), lambda i: (i, 0)),
        out_shape=jax.ShapeDtypeStruct((N, D), jnp.float32),
    )(h, W)


_mesh = plsc.VectorSubcoreMesh(core_axis_name="c", subcore_axis_name="s")


@functools.partial(
    pl.kernel,
    out_type=jax.ShapeDtypeStruct((NC, NP, D), jnp.float32),
    mesh=_mesh,
    scratch_types=[
        pltpu.VMEM((SB, CHUNK), jnp.int32),
        pltpu.VMEM((SB, CHUNK), jnp.int32),
        pltpu.VMEM((SB, CHUNK), jnp.float32),
        pltpu.VMEM((CHUNK, D), jnp.float32),
        pltpu.VMEM((CHUNK, D), jnp.float32),
        pltpu.VMEM_SHARED((NP, D), jnp.float32),
        pltpu.SemaphoreType.DMA,
        pltpu.SemaphoreType.DMA,
        pltpu.SemaphoreType.DMA,
        pltpu.SemaphoreType.DMA,
        pltpu.SemaphoreType.DMA,
        pltpu.SemaphoreType.DMA,
    ],
)
def _edge_kernel(hlin, row4, col4, w4, out, rowv, colv, wv,
                 gbufA, gbufB, acc, semgA, semgA2, semgB, semgB2, semsA, semsB):
    c = lax.axis_index("c")
    s = lax.axis_index("s")
    wid = c * NS + s

    H = CHUNK // 2

    def drain(sem, buf):
        pltpu.make_async_copy(hlin.at[pl.ds(0, CHUNK)], buf, sem).wait()

    def drain_half(sem, buf):
        pltpu.make_async_copy(
            hlin.at[pl.ds(0, H)], buf.at[pl.ds(0, H)], sem).wait()

    def gather2(k, buf, sem_lo, sem_hi):
        pltpu.async_copy(
            hlin.at[colv.at[k, pl.ds(0, H)]], buf.at[pl.ds(0, H)], sem_lo)
        pltpu.async_copy(
            hlin.at[colv.at[k, pl.ds(H, H)]], buf.at[pl.ds(H, H)], sem_hi)

    zero = jnp.zeros((LANES,), jnp.float32)

    def zbody(r, _):
        for j in range(D // LANES):
            gbufA[r, pl.ds(j * LANES, LANES)] = zero
        return 0

    lax.fori_loop(0, CHUNK, zbody, 0)
    for i in range(RPT // CHUNK):
        pltpu.sync_copy(gbufA, acc.at[pl.ds(s * RPT + i * CHUNK, CHUNK)])
    plsc.subcore_barrier()

    def scale(buf, k):
        for g in range(CHUNK // LANES):
            wvec = wv[k, pl.ds(g * LANES, LANES)]
            for i in range(LANES):
                w = wvec[i]
                e = g * LANES + i
                for j in range(D // LANES):
                    buf[e, pl.ds(j * LANES, LANES)] = (
                        buf[e, pl.ds(j * LANES, LANES)] * w)

    def step(k, X, gX, gX2, sX, Y, gY, gY2, sY):
        @pl.when(k >= 1)
        def _():
            drain(sY, Y)

        @pl.when(k <= SB - 2)
        def _():
            gather2(k + 1, Y, gY, gY2)

        drain_half(gX, X)
        drain_half(gX2, X)
        scale(X, k)
        pltpu.async_copy(X, acc.at[rowv.at[k]], sX, add=True)

    def sblock(b, _):
        pltpu.sync_copy(col4.at[wid, b], colv)
        gather2(0, gbufA, semgA, semgA2)
        pltpu.sync_copy(row4.at[wid, b], rowv)
        pltpu.sync_copy(w4.at[wid, b], wv)

        def chunk(k, _):
            @pl.when(k % 2 == 0)
            def _():
                step(k, gbufA, semgA, semgA2, semsA, gbufB, semgB, semgB2, semsB)

            @pl.when(k % 2 == 1)
            def _():
                step(k, gbufB, semgB, semgB2, semsB, gbufA, semgA, semgA2, semsA)

            return 0

        lax.fori_loop(0, SB, chunk, 0)
        drain(semsA, gbufA)
        return 0

    lax.fori_loop(0, SBLK, sblock, 0)

    plsc.subcore_barrier()
    pltpu.sync_copy(acc.at[pl.ds(s * RPT, RPT)], out.at[c, pl.ds(s * RPT, RPT)])


def _finish_body(p_ref, h0_ref, g_ref, b_ref, o_ref):
    a = p_ref[0] + p_ref[1]
    mean = jnp.mean(a, axis=-1, keepdims=True)
    xc = a - mean
    var = jnp.mean(xc * xc, axis=-1, keepdims=True)
    y = xc * lax.rsqrt(var + 1e-5) * g_ref[...] + b_ref[...]
    y = jnp.maximum(y, 0.0)
    o_ref[...] = (1.0 - ALPHA) * y + ALPHA * h0_ref[...]


def _finish(partials, h0, gamma, beta):
    BM = 5000
    return pl.pallas_call(
        _finish_body,
        grid=(N // BM,),
        in_specs=[pl.BlockSpec((NC, BM, D), lambda i: (0, i, 0)),
                  pl.BlockSpec((BM, D), lambda i: (i, 0)),
                  pl.BlockSpec((1, D), lambda i: (0, 0)),
                  pl.BlockSpec((1, D), lambda i: (0, 0))],
        out_specs=pl.BlockSpec((BM, D), lambda i: (i, 0)),
        out_shape=jax.ShapeDtypeStruct((N, D), jnp.float32),
    )(partials, h0, gamma, beta)


def kernel(h, h0, row, col, norm_weight, W, gamma, beta):
    row4 = row.astype(jnp.int32).reshape(NW, SBLK, SB, CHUNK)
    col4 = col.astype(jnp.int32).reshape(NW, SBLK, SB, CHUNK)
    w4 = norm_weight.reshape(NW, SBLK, SB, CHUNK)
    h_lin = _matmul(h, W)
    partials = _edge_kernel(h_lin, row4, col4, w4)
    return _finish(partials, h0, gamma.reshape(1, D), beta.reshape(1, D))
